# baseline (speedup 1.0000x reference)
import functools

import jax
import jax.numpy as jnp
from jax import lax
from jax.experimental import pallas as pl
from jax.experimental.pallas import tpu as pltpu
from jax.experimental.pallas import tpu_sc as plsc

EMBED_DIM = 32
SCALE = float(EMBED_DIM) ** 0.5
LANES = 16

_NC = 2
_NS = 16
_NW = _NC * _NS

_G = 4
_TOK = _G * 128


def _make_sc_lookup(n_b, n_s, dim):
    n_bt = n_b // 128
    n_btg = n_bt // _G
    units = n_s * n_btg
    upw = units // _NW
    assert upw % 2 == 0 and dim == 32
    batch = n_b * n_s
    tok_pw = upw * _TOK

    mesh = plsc.VectorSubcoreMesh(core_axis_name="c", subcore_axis_name="s")

    @functools.partial(
        pl.kernel,
        mesh=mesh,
        out_type=jax.ShapeDtypeStruct((n_s, dim // 8, n_bt, 1024),
                                      jnp.float32),
        scratch_types=[
            pltpu.VMEM((tok_pw,), jnp.int32),
            pltpu.VMEM((_TOK, dim), jnp.float32),
            pltpu.VMEM((_TOK, dim), jnp.float32),
            pltpu.VMEM(((dim // 8) * _G, 1024), jnp.float32),
            pltpu.VMEM(((dim // 8) * _G, 1024), jnp.float32),
            pltpu.SemaphoreType.DMA,
            pltpu.SemaphoreType.DMA,
            pltpu.SemaphoreType.DMA,
            pltpu.SemaphoreType.DMA,
        ],
        compiler_params=pltpu.CompilerParams(use_tc_tiling_on_sc=False,
                                             needs_layout_passes=False),
    )
    def lookup(table_hbm, tokens_hbm, out_hbm, idx_v, rows0, rows1,
               ob0, ob1, g0, g1, s0, s1):
        wid = lax.axis_index("s") * _NC + lax.axis_index("c")
        ubase = wid * upw
        pltpu.sync_copy(tokens_hbm.at[pl.ds(ubase * _TOK, tok_pw)], idx_v)

        rows = (rows0, rows1)
        obuf = (ob0, ob1)
        gsem = (g0, g1)
        ssem = (s0, s1)

        row_iota = lax.iota(jnp.int32, LANES)
        col_vecs = []
        rowb_vecs = []
        colb_vecs = []
        for d0 in range(dim):
            cols = (row_iota + d0) % dim
            col_vecs.append(cols)
            rowb_vecs.append((cols // 8) * _G)
            colb_vecs.append((cols % 8) * 128 + row_iota)

        def start_gather(b, u):
            return pltpu.async_copy(
                table_hbm.at[idx_v.at[pl.ds(u * _TOK, _TOK)]],
                rows[b], gsem[b])

        def out_pieces(b, u):
            ug = ubase + u
            s = ug // n_btg
            btg = ug % n_btg
            return [(obuf[b].at[pl.ds(dt * _G, _G)],
                     out_hbm.at[s, dt, pl.ds(btg * _G, _G)])
                    for dt in range(dim // 8)]

        def process(b, u, wait_prev_store):
            pltpu.make_async_copy(
                table_hbm.at[idx_v.at[pl.ds(u * _TOK, _TOK)]],
                rows[b], gsem[b]).wait()
            if wait_prev_store:
                for src, dst in out_pieces(b, u - 2):
                    pltpu.make_async_copy(src, dst, ssem[b]).wait()


            def tr_body(m, carry):
                ridx = row_iota + m * LANES
                g = m // 8
                l0 = (m % 8) * LANES
                for d0 in range(dim):
                    vals = plsc.load_gather(rows[b], [ridx, col_vecs[d0]])
                    plsc.store_scatter(
                        obuf[b], [rowb_vecs[d0] + g, colb_vecs[d0] + l0],
                        vals * SCALE)
                return carry

            lax.fori_loop(0, _TOK // LANES, tr_body, 0, unroll=2)
            for src, dst in out_pieces(b, u):
                pltpu.async_copy(src, dst, ssem[b])
            nxt = jnp.minimum(u + 2, upw - 1)
            start_gather(b, nxt)

        start_gather(0, 0)
        start_gather(1, 1)
        process(0, 0, False)
        process(1, 1, False)

        def pair_body(up, carry):
            u = up * 2
            process(0, u, True)
            process(1, u + 1, True)
            return carry

        lax.fori_loop(1, upw // 2, pair_body, 0)

        for b in (0, 1):
            pltpu.make_async_copy(
                table_hbm.at[idx_v.at[pl.ds((upw - 1) * _TOK, _TOK)]],
                rows[b], gsem[b]).wait()
            for src, dst in out_pieces(b, upw - 2 + b):
                pltpu.make_async_copy(src, dst, ssem[b]).wait()

    return lookup


_VC = 768


def _make_table_relayout(vocab, dim):
    full = (vocab // _VC) * _VC
    n_ch = vocab // _VC
    tail = vocab - full
    kmax = -(-n_ch // _NW)

    mesh = plsc.VectorSubcoreMesh(core_axis_name="c", subcore_axis_name="s")

    @functools.partial(
        pl.kernel,
        mesh=mesh,
        out_type=jax.ShapeDtypeStruct((vocab * dim // 128, 128), jnp.float32),
        scratch_types=[
            pltpu.VMEM((dim, _VC), jnp.float32),
            pltpu.VMEM((dim, _VC), jnp.float32),
            pltpu.VMEM((_VC * dim // 128, 128), jnp.float32),
            pltpu.VMEM((_VC * dim // 128, 128), jnp.float32),
            pltpu.SemaphoreType.DMA,
            pltpu.SemaphoreType.DMA,
            pltpu.SemaphoreType.DMA,
            pltpu.SemaphoreType.DMA,
        ],
        compiler_params=pltpu.CompilerParams(needs_layout_passes=False),
    )
    def relayout(tt_hbm, tail_hbm, out_hbm, in0, in1, ob0, ob1,
                 g0, g1, s0, s1):
        wid = lax.axis_index("s") * _NC + lax.axis_index("c")

        ibuf = (in0, in1)
        obuf = (ob0, ob1)
        gsem = (g0, g1)
        ssem = (s0, s1)

        row_iota = lax.iota(jnp.int32, LANES)
        col_vecs = []
        orow_vecs = []
        olane_vecs = []
        for d0 in range(dim):
            cv = (row_iota + d0) % dim
            fidx0 = row_iota * dim + cv
            col_vecs.append(cv)
            orow_vecs.append(fidx0 // 128)
            olane_vecs.append(fidx0 % 128)

        def chunk_id(k):
            return jnp.minimum(wid + k * _NW, n_ch - 1)

        def start_stage(b, k):
            c = chunk_id(k)
            return pltpu.async_copy(
                tt_hbm.at[:, pl.ds(c * _VC, _VC)], ibuf[b], gsem[b])

        def out_slice(c):
            return out_hbm.at[pl.ds(c * (_VC * dim // 128), _VC * dim // 128)]

        def process(b, k, wait_prev_store):
            c = chunk_id(k)
            pltpu.make_async_copy(
                tt_hbm.at[:, pl.ds(c * _VC, _VC)], ibuf[b], gsem[b]).wait()
            if wait_prev_store:
                pltpu.make_async_copy(obuf[b], out_slice(chunk_id(k - 2)),
                                      ssem[b]).wait()

            def tr_body(m, carry):
                t0 = m * LANES
                ridx = row_iota + t0
                r0 = m * (LANES * dim // 128)
                for d0 in range(dim):
                    vals = plsc.load_gather(ibuf[b], [col_vecs[d0], ridx])
                    plsc.store_scatter(
                        obuf[b], [orow_vecs[d0] + r0, olane_vecs[d0]], vals)
                return carry

            lax.fori_loop(0, _VC // LANES, tr_body, 0)
            pltpu.async_copy(obuf[b], out_slice(c), ssem[b])
            start_stage(b, k + 2)

        start_stage(0, 0)
        start_stage(1, 1)
        process(0, 0, False)
        process(1, 1, False)

        def pair_body(kp, carry):
            process(0, kp * 2, True)
            process(1, kp * 2 + 1, True)
            return carry

        lax.fori_loop(1, (kmax + 1) // 2, pair_body, 0)

        for b in (0, 1):
            pltpu.make_async_copy(
                tt_hbm.at[:, pl.ds((n_ch - 1) * _VC, _VC)],
                ibuf[b], gsem[b]).wait()
            kk = 2 * ((kmax + 1) // 2) - 2 + b
            pltpu.make_async_copy(obuf[b], out_slice(chunk_id(kk)),
                                  ssem[b]).wait()

        @pl.when(wid == 0)
        def _():
            pltpu.sync_copy(
                tail_hbm,
                out_hbm.at[pl.ds(full * dim // 128, tail * dim // 128)])

    return relayout


def kernel(tokens, table):
    n_b, n_s = tokens.shape
    vocab, dim = table.shape
    batch = n_b * n_s
    flat_t = tokens.T.reshape(batch).astype(jnp.int32)
    full = (vocab // _VC) * _VC
    tail2d = table[full:].reshape((vocab - full) * dim // 128, 128)
    table_lin = _make_table_relayout(vocab, dim)(table.T, tail2d)
    table_rm = table_lin.reshape(vocab, dim)
    out4 = _make_sc_lookup(n_b, n_s, EMBED_DIM)(table_rm, flat_t)
    out5 = out4.reshape(n_s, EMBED_DIM // 8, n_b // 128, 8, 128)
    return out5.transpose(2, 4, 0, 1, 3).reshape(n_b, n_s, EMBED_DIM)


# speedup vs baseline: 1.0383x; 1.0383x over previous
import functools

import jax
import jax.numpy as jnp
from jax import lax
from jax.experimental import pallas as pl
from jax.experimental.pallas import tpu as pltpu
from jax.experimental.pallas import tpu_sc as plsc

EMBED_DIM = 32
SCALE = float(EMBED_DIM) ** 0.5
LANES = 16

_NC = 2
_NS = 16
_NW = _NC * _NS

_G = 4
_TOK = _G * 128


def _make_sc_lookup(n_b, n_s, dim):
    n_bt = n_b // 128
    n_btg = n_bt // _G
    units = n_s * n_btg
    upw = units // _NW
    assert upw % 2 == 0 and dim == 32
    batch = n_b * n_s
    tok_pw = upw * _TOK

    mesh = plsc.VectorSubcoreMesh(core_axis_name="c", subcore_axis_name="s")

    @functools.partial(
        pl.kernel,
        mesh=mesh,
        out_type=jax.ShapeDtypeStruct((n_s, dim // 8, n_bt, 1024),
                                      jnp.float32),
        scratch_types=[
            pltpu.VMEM((tok_pw,), jnp.int32),
            pltpu.VMEM((_TOK, dim), jnp.float32),
            pltpu.VMEM((_TOK, dim), jnp.float32),
            pltpu.VMEM(((dim // 8) * _G, 1024), jnp.float32),
            pltpu.VMEM(((dim // 8) * _G, 1024), jnp.float32),
            pltpu.SemaphoreType.DMA,
            pltpu.SemaphoreType.DMA,
            pltpu.SemaphoreType.DMA,
            pltpu.SemaphoreType.DMA,
        ],
        compiler_params=pltpu.CompilerParams(use_tc_tiling_on_sc=False,
                                             needs_layout_passes=False),
    )
    def lookup(table_hbm, tokens_hbm, out_hbm, idx_v, rows0, rows1,
               ob0, ob1, g0, g1, s0, s1):
        wid = lax.axis_index("s") * _NC + lax.axis_index("c")
        ubase = wid * upw
        pltpu.sync_copy(tokens_hbm.at[pl.ds(ubase * _TOK, tok_pw)], idx_v)

        rows = (rows0, rows1)
        obuf = (ob0, ob1)
        gsem = (g0, g1)
        ssem = (s0, s1)

        row_iota = lax.iota(jnp.int32, LANES)
        col_vecs = []
        rowb_vecs = []
        colb_vecs = []
        for d0 in range(dim):
            cols = (row_iota + d0) % dim
            col_vecs.append(cols)
            rowb_vecs.append((cols // 8) * _G)
            colb_vecs.append((cols % 8) * 128 + row_iota)

        def start_gather(b, u):
            return pltpu.async_copy(
                table_hbm.at[idx_v.at[pl.ds(u * _TOK, _TOK)]],
                rows[b], gsem[b])

        def out_pieces(b, u):
            ug = ubase + u
            s = ug // n_btg
            btg = ug % n_btg
            return [(obuf[b].at[pl.ds(dt * _G, _G)],
                     out_hbm.at[s, dt, pl.ds(btg * _G, _G)])
                    for dt in range(dim // 8)]

        def process(b, u, wait_prev_store):
            pltpu.make_async_copy(
                table_hbm.at[idx_v.at[pl.ds(u * _TOK, _TOK)]],
                rows[b], gsem[b]).wait()
            if wait_prev_store:
                for src, dst in out_pieces(b, u - 2):
                    pltpu.make_async_copy(src, dst, ssem[b]).wait()


            def tr_body(m, carry):
                ridx = row_iota + m * LANES
                g = m // 8
                l0 = (m % 8) * LANES
                for d0 in range(dim):
                    vals = plsc.load_gather(rows[b], [ridx, col_vecs[d0]])
                    plsc.store_scatter(
                        obuf[b], [rowb_vecs[d0] + g, colb_vecs[d0] + l0],
                        vals * SCALE)
                return carry

            lax.fori_loop(0, _TOK // LANES, tr_body, 0)
            for src, dst in out_pieces(b, u):
                pltpu.async_copy(src, dst, ssem[b])
            nxt = jnp.minimum(u + 2, upw - 1)
            start_gather(b, nxt)

        start_gather(0, 0)
        start_gather(1, 1)
        process(0, 0, False)
        process(1, 1, False)

        def pair_body(up, carry):
            u = up * 2
            process(0, u, True)
            process(1, u + 1, True)
            return carry

        lax.fori_loop(1, upw // 2, pair_body, 0)

        for b in (0, 1):
            pltpu.make_async_copy(
                table_hbm.at[idx_v.at[pl.ds((upw - 1) * _TOK, _TOK)]],
                rows[b], gsem[b]).wait()
            for src, dst in out_pieces(b, upw - 2 + b):
                pltpu.make_async_copy(src, dst, ssem[b]).wait()

    return lookup


_VC = 768


def _make_table_relayout(vocab, dim):
    full = (vocab // _VC) * _VC
    n_ch = vocab // _VC
    tail = vocab - full
    kmax = -(-n_ch // _NW)

    mesh = plsc.VectorSubcoreMesh(core_axis_name="c", subcore_axis_name="s")

    @functools.partial(
        pl.kernel,
        mesh=mesh,
        out_type=jax.ShapeDtypeStruct((vocab * dim // 128, 128), jnp.float32),
        scratch_types=[
            pltpu.VMEM((dim, _VC), jnp.float32),
            pltpu.VMEM((dim, _VC), jnp.float32),
            pltpu.VMEM((_VC * dim // 128, 128), jnp.float32),
            pltpu.VMEM((_VC * dim // 128, 128), jnp.float32),
            pltpu.SemaphoreType.DMA,
            pltpu.SemaphoreType.DMA,
            pltpu.SemaphoreType.DMA,
            pltpu.SemaphoreType.DMA,
        ],
        compiler_params=pltpu.CompilerParams(needs_layout_passes=False),
    )
    def relayout(tt_hbm, tail_hbm, out_hbm, in0, in1, ob0, ob1,
                 g0, g1, s0, s1):
        wid = lax.axis_index("s") * _NC + lax.axis_index("c")

        ibuf = (in0, in1)
        obuf = (ob0, ob1)
        gsem = (g0, g1)
        ssem = (s0, s1)

        row_iota = lax.iota(jnp.int32, LANES)
        col_vecs = []
        orow_vecs = []
        olane_vecs = []
        for d0 in range(dim):
            cv = (row_iota + d0) % dim
            fidx0 = row_iota * dim + cv
            col_vecs.append(cv)
            orow_vecs.append(fidx0 // 128)
            olane_vecs.append(fidx0 % 128)

        def chunk_id(k):
            return jnp.minimum(wid + k * _NW, n_ch - 1)

        def start_stage(b, k):
            c = chunk_id(k)
            return pltpu.async_copy(
                tt_hbm.at[:, pl.ds(c * _VC, _VC)], ibuf[b], gsem[b])

        def out_slice(c):
            return out_hbm.at[pl.ds(c * (_VC * dim // 128), _VC * dim // 128)]

        def process(b, k, wait_prev_store):
            c = chunk_id(k)
            pltpu.make_async_copy(
                tt_hbm.at[:, pl.ds(c * _VC, _VC)], ibuf[b], gsem[b]).wait()
            if wait_prev_store:
                pltpu.make_async_copy(obuf[b], out_slice(chunk_id(k - 2)),
                                      ssem[b]).wait()

            def tr_body(m, carry):
                t0 = m * LANES
                ridx = row_iota + t0
                r0 = m * (LANES * dim // 128)
                for d0 in range(dim):
                    vals = plsc.load_gather(ibuf[b], [col_vecs[d0], ridx])
                    plsc.store_scatter(
                        obuf[b], [orow_vecs[d0] + r0, olane_vecs[d0]], vals)
                return carry

            lax.fori_loop(0, _VC // LANES, tr_body, 0)
            pltpu.async_copy(obuf[b], out_slice(c), ssem[b])
            start_stage(b, k + 2)

        start_stage(0, 0)
        start_stage(1, 1)
        process(0, 0, False)
        process(1, 1, False)

        def pair_body(kp, carry):
            process(0, kp * 2, True)
            process(1, kp * 2 + 1, True)
            return carry

        lax.fori_loop(1, (kmax + 1) // 2, pair_body, 0)

        for b in (0, 1):
            pltpu.make_async_copy(
                tt_hbm.at[:, pl.ds((n_ch - 1) * _VC, _VC)],
                ibuf[b], gsem[b]).wait()
            kk = 2 * ((kmax + 1) // 2) - 2 + b
            pltpu.make_async_copy(obuf[b], out_slice(chunk_id(kk)),
                                  ssem[b]).wait()

        @pl.when(wid == 0)
        def _():
            pltpu.sync_copy(
                tail_hbm,
                out_hbm.at[pl.ds(full * dim // 128, tail * dim // 128)])

    return relayout


def kernel(tokens, table):
    n_b, n_s = tokens.shape
    vocab, dim = table.shape
    batch = n_b * n_s
    flat_t = tokens.T.reshape(batch).astype(jnp.int32)
    full = (vocab // _VC) * _VC
    tail2d = table[full:].reshape((vocab - full) * dim // 128, 128)
    table_lin = _make_table_relayout(vocab, dim)(table.T, tail2d)
    table_rm = table_lin.reshape(vocab, dim)
    out4 = _make_sc_lookup(n_b, n_s, EMBED_DIM)(table_rm, flat_t)
    out5 = out4.reshape(n_s, EMBED_DIM // 8, n_b // 128, 8, 128)
    return out5.transpose(2, 4, 0, 1, 3).reshape(n_b, n_s, EMBED_DIM)
